# baseline (speedup 1.0000x reference)
import jax
import jax.numpy as jnp
from jax import lax
from jax.experimental import pallas as pl
from jax.experimental.pallas import tpu as pltpu
from jax.experimental.pallas import tpu_sc as plsc

NC = 2
NS = 16
LANES = 16
NW = NC * NS

D = 32
CHUNK = 128
WPAD = 17


def _mf_body(user_ref, item_ref, ut_ref, it_ref, out_ref,
             uidx, iidx, urows, vrows, outv, sem):
    n_chunks, chunk = uidx.shape
    b_per_w = n_chunks * chunk
    n_groups = b_per_w // LANES

    wid = lax.axis_index("s") * NC + lax.axis_index("c")
    base = wid * b_per_w

    stages = []
    for c in range(n_chunks):
        src = pl.ds(base + c * chunk, chunk)
        stages.append(pltpu.async_copy(user_ref.at[src], uidx.at[c], sem))
        stages.append(pltpu.async_copy(item_ref.at[src], iidx.at[c], sem))
    for s in stages:
        s.wait()

    gathers = []
    for c in range(n_chunks):
        gathers.append(pltpu.async_copy(ut_ref.at[uidx.at[c]], urows.at[c], sem))
        gathers.append(pltpu.async_copy(it_ref.at[iidx.at[c]], vrows.at[c], sem))
    for g in gathers:
        g.wait()

    lane = lax.iota(jnp.int32, LANES)
    last = jnp.full((LANES,), LANES - 1, jnp.int32)

    groups_per_chunk = chunk // LANES

    for c in range(n_chunks):
        def grp_body(g, _, c=c):
            base_r = g * LANES
            acc = jnp.zeros((LANES,), jnp.float32)
            for j in range(LANES):
                r = base_r + j
                u0 = urows[c, r, pl.ds(0, LANES)]
                u1 = urows[c, r, pl.ds(LANES, LANES)]
                v0 = vrows[c, r, pl.ds(0, LANES)]
                v1 = vrows[c, r, pl.ds(LANES, LANES)]
                w = u0 * v0 + u1 * v1
                tot = jnp.take(plsc.cumsum(w), last)
                acc = jnp.where(lane == j, tot, acc)
            outv[pl.ds(c * chunk + base_r, LANES)] = acc
            return 0

        lax.fori_loop(0, groups_per_chunk, grp_body, 0)

    pltpu.sync_copy(outv, out_ref.at[pl.ds(base, b_per_w)])


def _build(batch):
    b_per_w = batch // NW
    n_chunks = b_per_w // CHUNK
    mesh = plsc.VectorSubcoreMesh(core_axis_name="c", subcore_axis_name="s")
    return pl.kernel(
        _mf_body,
        out_type=jax.ShapeDtypeStruct((batch,), jnp.float32),
        mesh=mesh,
        compiler_params=pltpu.CompilerParams(
            needs_layout_passes=False, use_tc_tiling_on_sc=False),
        scratch_types=[
            pltpu.VMEM((n_chunks, CHUNK), jnp.int32),
            pltpu.VMEM((n_chunks, CHUNK), jnp.int32),
            pltpu.VMEM((n_chunks, CHUNK, D), jnp.float32),
            pltpu.VMEM((n_chunks, CHUNK, D), jnp.float32),
            pltpu.VMEM((b_per_w,), jnp.float32),
            pltpu.SemaphoreType.DMA,
        ],
    )


@jax.jit
def kernel(user, item, user_table, item_table):
    batch = user.shape[0]
    return _build(batch)(user, item, user_table, item_table)
